# Initial kernel scaffold; baseline (speedup 1.0000x reference)
import jax
import jax.numpy as jnp
from jax.experimental import pallas as pl


def kernel(cls_logits, bbox_pred, priors):
    raise NotImplementedError("write your pallas kernel here")


# speedup vs baseline: 7.7258x; 7.7258x over previous
import functools

import jax
import jax.numpy as jnp
from jax.experimental import pallas as pl

IMAGE_SIZE = 512.0
CONFIDENCE_THRESHOLD = 0.01
NMS_THRESHOLD = 0.45
MAX_PER_IMAGE = 100
CENTER_VARIANCE = 0.1
SIZE_VARIANCE = 0.2
NEG = -1e30
CLASS_OFFSET = 4.0 * IMAGE_SIZE

N_PRIORS = 20000
N_CLASSES = 21
N_KEEP_PAD = 128


def _nms_kernel(logits_ref, loc_ref, prior_ref,
                sc_ref, lb_ref, x1_ref, y1_ref, x2_ref, y2_ref):
    C = N_CLASSES - 1

    logits = logits_ref[...]
    m = jnp.max(logits, axis=0, keepdims=True)
    e = jnp.exp(logits - m)
    denom = jnp.sum(e, axis=0, keepdims=True)
    probs = e / denom
    scores = probs[1:, :]

    loc = loc_ref[...]
    pr = prior_ref[...]
    cx = loc[0:1, :] * CENTER_VARIANCE * pr[2:3, :] + pr[0:1, :]
    cy = loc[1:2, :] * CENTER_VARIANCE * pr[3:4, :] + pr[1:2, :]
    w = jnp.exp(loc[2:3, :] * SIZE_VARIANCE) * pr[2:3, :]
    h = jnp.exp(loc[3:4, :] * SIZE_VARIANCE) * pr[3:4, :]
    x1 = (cx - w * 0.5) * IMAGE_SIZE
    y1 = (cy - h * 0.5) * IMAGE_SIZE
    x2 = (cx + w * 0.5) * IMAGE_SIZE
    y2 = (cy + h * 0.5) * IMAGE_SIZE
    area = (x2 - x1) * (y2 - y1)

    S = jnp.where(scores > CONFIDENCE_THRESHOLD, scores, NEG)

    row = jax.lax.broadcasted_iota(jnp.int32, (C, N_PRIORS), 0)
    col = jax.lax.broadcasted_iota(jnp.int32, (C, N_PRIORS), 1)
    fidx = col * C + row
    lane = jax.lax.broadcasted_iota(jnp.int32, (1, N_KEEP_PAD), 1)

    row_off = (jax.lax.broadcasted_iota(jnp.int32, (C, 1), 0).astype(jnp.float32)
               + 1.0) * CLASS_OFFSET

    def body(k, carry):
        S, acc_s, acc_l, acc_x1, acc_y1, acc_x2, acc_y2 = carry
        best_val = jnp.max(S)
        best_valid = best_val > -1e20
        fbest = jnp.min(jnp.where(S == best_val, fidx, jnp.int32(2**30)))
        p_best = fbest // C
        r_best = fbest % C

        pmask = col[0:1, :] == p_best
        bx1 = jnp.max(jnp.where(pmask, x1, NEG))
        by1 = jnp.max(jnp.where(pmask, y1, NEG))
        bx2 = jnp.max(jnp.where(pmask, x2, NEG))
        by2 = jnp.max(jnp.where(pmask, y2, NEG))
        barea = (bx2 - bx1) * (by2 - by1)

        d = (jnp.float32(r_best) + 1.0) * CLASS_OFFSET - row_off
        ix1 = jnp.maximum(bx1 + d, x1)
        iy1 = jnp.maximum(by1 + d, y1)
        ix2 = jnp.minimum(bx2 + d, x2)
        iy2 = jnp.minimum(by2 + d, y2)
        iw = jnp.maximum(ix2 - ix1, 0.0)
        ih = jnp.maximum(iy2 - iy1, 0.0)
        inter = iw * ih
        iou = inter / (barea + area - inter + 1e-9)

        S = jnp.where((iou > NMS_THRESHOLD) | (fidx == fbest), NEG, S)

        at_k = lane == k
        acc_s = jnp.where(at_k, jnp.where(best_valid, best_val, 0.0), acc_s)
        acc_l = jnp.where(at_k, jnp.where(best_valid, r_best + 1, 0), acc_l)
        zf = jnp.float32(0.0)
        acc_x1 = jnp.where(at_k, jnp.where(best_valid, bx1, zf), acc_x1)
        acc_y1 = jnp.where(at_k, jnp.where(best_valid, by1, zf), acc_y1)
        acc_x2 = jnp.where(at_k, jnp.where(best_valid, bx2, zf), acc_x2)
        acc_y2 = jnp.where(at_k, jnp.where(best_valid, by2, zf), acc_y2)
        return (S, acc_s, acc_l, acc_x1, acc_y1, acc_x2, acc_y2)

    zero_f = jnp.zeros((1, N_KEEP_PAD), jnp.float32)
    zero_i = jnp.zeros((1, N_KEEP_PAD), jnp.int32)
    carry = (S, zero_f, zero_i, zero_f, zero_f, zero_f, zero_f)
    carry = jax.lax.fori_loop(0, MAX_PER_IMAGE, body, carry)
    _, acc_s, acc_l, acc_x1, acc_y1, acc_x2, acc_y2 = carry

    sc_ref[...] = acc_s
    lb_ref[...] = acc_l
    x1_ref[...] = acc_x1
    y1_ref[...] = acc_y1
    x2_ref[...] = acc_x2
    y2_ref[...] = acc_y2


@jax.jit
def kernel(cls_logits, bbox_pred, priors):
    logits_t = jnp.transpose(cls_logits[0])
    loc_t = jnp.transpose(bbox_pred[0])
    prior_t = jnp.transpose(priors)

    out_shapes = (
        jax.ShapeDtypeStruct((1, N_KEEP_PAD), jnp.float32),
        jax.ShapeDtypeStruct((1, N_KEEP_PAD), jnp.int32),
        jax.ShapeDtypeStruct((1, N_KEEP_PAD), jnp.float32),
        jax.ShapeDtypeStruct((1, N_KEEP_PAD), jnp.float32),
        jax.ShapeDtypeStruct((1, N_KEEP_PAD), jnp.float32),
        jax.ShapeDtypeStruct((1, N_KEEP_PAD), jnp.float32),
    )
    sc, lb, x1, y1, x2, y2 = pl.pallas_call(
        _nms_kernel,
        out_shape=out_shapes,
    )(logits_t, loc_t, prior_t)

    out_scores = sc[0, :MAX_PER_IMAGE]
    out_labels = lb[0, :MAX_PER_IMAGE]
    out_boxes = jnp.stack(
        [x1[0, :MAX_PER_IMAGE], y1[0, :MAX_PER_IMAGE],
         x2[0, :MAX_PER_IMAGE], y2[0, :MAX_PER_IMAGE]], axis=-1)
    return out_boxes, out_scores, out_labels


# speedup vs baseline: 27.7413x; 3.5907x over previous
import functools

import jax
import jax.numpy as jnp
from jax.experimental import pallas as pl
from jax.experimental.pallas import tpu as pltpu

IMAGE_SIZE = 512.0
CONFIDENCE_THRESHOLD = 0.01
NMS_THRESHOLD = 0.45
MAX_PER_IMAGE = 100
CENTER_VARIANCE = 0.1
SIZE_VARIANCE = 0.2
NEG = -1e30
CLASS_OFFSET = 4.0 * IMAGE_SIZE

N_PRIORS = 20000
N_CLASSES = 21
C = N_CLASSES - 1
N_BLOCKS = 157
N_PAD = N_BLOCKS * 128
N_KEEP_PAD = 128
BIG_I = 2**30


def _nms_kernel(logits_ref, loc_ref, prior_ref,
                sc_ref, lb_ref, x1_ref, y1_ref, x2_ref, y2_ref,
                cand_ref, colmax_ref):
    logits = logits_ref[...]
    m = jnp.max(logits, axis=0, keepdims=True)
    e = jnp.exp(logits - m)
    probs = e / jnp.sum(e, axis=0, keepdims=True)
    scores = probs[1:, :]

    loc = loc_ref[...]
    pr = prior_ref[...]
    cx = loc[0:1, :] * CENTER_VARIANCE * pr[2:3, :] + pr[0:1, :]
    cy = loc[1:2, :] * CENTER_VARIANCE * pr[3:4, :] + pr[1:2, :]
    w = jnp.exp(loc[2:3, :] * SIZE_VARIANCE) * pr[2:3, :]
    h = jnp.exp(loc[3:4, :] * SIZE_VARIANCE) * pr[3:4, :]
    x1 = (cx - w * 0.5) * IMAGE_SIZE
    y1 = (cy - h * 0.5) * IMAGE_SIZE
    x2 = (cx + w * 0.5) * IMAGE_SIZE
    y2 = (cy + h * 0.5) * IMAGE_SIZE

    S = jnp.where(scores > CONFIDENCE_THRESHOLD, scores, NEG)

    comb = jnp.concatenate(
        [S, x1, y1, x2, y2, jnp.full((8, N_PRIORS), NEG, jnp.float32)], axis=0)
    cand_ref[...] = jnp.transpose(comb)

    colmax_flat = jnp.max(S, axis=0, keepdims=True)
    colmax_flat = jnp.concatenate(
        [colmax_flat, jnp.full((1, N_PAD - N_PRIORS), NEG, jnp.float32)],
        axis=1)
    cm2d = jnp.reshape(colmax_flat, (N_BLOCKS, 128))
    colmax_ref[...] = cm2d

    rowmax = jnp.max(cm2d, axis=1, keepdims=True)
    rowmax_t = jnp.transpose(rowmax)
    rowmax_t = jnp.concatenate(
        [rowmax_t, jnp.full((1, 1024 - N_BLOCKS), NEG, jnp.float32)], axis=1)
    bm0 = jnp.reshape(rowmax_t, (8, 128))

    lane32 = jax.lax.broadcasted_iota(jnp.int32, (1, 32), 1)
    lane128 = jax.lax.broadcasted_iota(jnp.int32, (1, N_KEEP_PAD), 1)
    bflat = jax.lax.broadcasted_iota(jnp.int32, (8, 128), 0) * 128 + \
        jax.lax.broadcasted_iota(jnp.int32, (8, 128), 1)

    def cond(carry):
        nkeep, bm = carry[0], carry[1]
        return (nkeep < MAX_PER_IMAGE) & (jnp.max(bm) > -1e20)

    def body(carry):
        (nkeep, bm, kox1, koy1, kox2, koy2, karea,
         acc_s, acc_l, acc_x1, acc_y1, acc_x2, acc_y2) = carry

        gmax = jnp.max(bm)
        b = jnp.min(jnp.where(bm == gmax, bflat, BIG_I))
        crow = colmax_ref[pl.ds(b, 1), :]
        j = jnp.min(jnp.where(crow == gmax, lane128, BIG_I))
        p = b * 128 + j

        row = cand_ref[pl.ds(p, 1), :]
        is_score = lane32 < C
        r = jnp.min(jnp.where(is_score & (row == gmax), lane32, BIG_I))

        bx1 = jnp.max(jnp.where(lane32 == C, row, NEG))
        by1 = jnp.max(jnp.where(lane32 == C + 1, row, NEG))
        bx2 = jnp.max(jnp.where(lane32 == C + 2, row, NEG))
        by2 = jnp.max(jnp.where(lane32 == C + 3, row, NEG))

        off = (r + 1).astype(jnp.float32) * CLASS_OFFSET
        bxo1 = bx1 + off
        byo1 = by1 + off
        bxo2 = bx2 + off
        byo2 = by2 + off
        barea = (bxo2 - bxo1) * (byo2 - byo1)

        iw = jnp.maximum(jnp.minimum(kox2, bxo2) - jnp.maximum(kox1, bxo1), 0.0)
        ih = jnp.maximum(jnp.minimum(koy2, byo2) - jnp.maximum(koy1, byo1), 0.0)
        inter = iw * ih
        iou = inter / (karea + barea - inter + 1e-9)
        supp = jnp.any((iou > NMS_THRESHOLD) & (lane128 < nkeep))

        at_k = (lane128 == nkeep) & jnp.logical_not(supp)
        kox1 = jnp.where(at_k, bxo1, kox1)
        koy1 = jnp.where(at_k, byo1, koy1)
        kox2 = jnp.where(at_k, bxo2, kox2)
        koy2 = jnp.where(at_k, byo2, koy2)
        karea = jnp.where(at_k, barea, karea)
        acc_s = jnp.where(at_k, gmax, acc_s)
        acc_l = jnp.where(at_k, r + 1, acc_l)
        acc_x1 = jnp.where(at_k, bx1, acc_x1)
        acc_y1 = jnp.where(at_k, by1, acc_y1)
        acc_x2 = jnp.where(at_k, bx2, acc_x2)
        acc_y2 = jnp.where(at_k, by2, acc_y2)
        nkeep = nkeep + jnp.where(supp, 0, 1)

        row_new = jnp.where(lane32 == r, NEG, row)
        cand_ref[pl.ds(p, 1), :] = row_new
        head = jnp.max(jnp.where(is_score, row_new, NEG))
        crow_new = jnp.where(lane128 == j, head, crow)
        colmax_ref[pl.ds(b, 1), :] = crow_new
        bm = jnp.where(bflat == b, jnp.max(crow_new), bm)

        return (nkeep, bm, kox1, koy1, kox2, koy2, karea,
                acc_s, acc_l, acc_x1, acc_y1, acc_x2, acc_y2)

    zf = jnp.zeros((1, N_KEEP_PAD), jnp.float32)
    zi = jnp.zeros((1, N_KEEP_PAD), jnp.int32)
    carry = (jnp.int32(0), bm0, zf, zf, zf, zf, zf,
             zf, zi, zf, zf, zf, zf)
    carry = jax.lax.while_loop(cond, body, carry)
    (_, _, _, _, _, _, _,
     acc_s, acc_l, acc_x1, acc_y1, acc_x2, acc_y2) = carry

    sc_ref[...] = acc_s
    lb_ref[...] = acc_l
    x1_ref[...] = acc_x1
    y1_ref[...] = acc_y1
    x2_ref[...] = acc_x2
    y2_ref[...] = acc_y2


@jax.jit
def kernel(cls_logits, bbox_pred, priors):
    logits_t = jnp.transpose(cls_logits[0])
    loc_t = jnp.transpose(bbox_pred[0])
    prior_t = jnp.transpose(priors)

    out_shapes = (
        jax.ShapeDtypeStruct((1, N_KEEP_PAD), jnp.float32),
        jax.ShapeDtypeStruct((1, N_KEEP_PAD), jnp.int32),
        jax.ShapeDtypeStruct((1, N_KEEP_PAD), jnp.float32),
        jax.ShapeDtypeStruct((1, N_KEEP_PAD), jnp.float32),
        jax.ShapeDtypeStruct((1, N_KEEP_PAD), jnp.float32),
        jax.ShapeDtypeStruct((1, N_KEEP_PAD), jnp.float32),
    )
    sc, lb, x1, y1, x2, y2 = pl.pallas_call(
        _nms_kernel,
        out_shape=out_shapes,
        scratch_shapes=[
            pltpu.VMEM((N_PRIORS, 32), jnp.float32),
            pltpu.VMEM((N_BLOCKS, 128), jnp.float32),
        ],
    )(logits_t, loc_t, prior_t)

    out_scores = sc[0, :MAX_PER_IMAGE]
    out_labels = lb[0, :MAX_PER_IMAGE]
    out_boxes = jnp.stack(
        [x1[0, :MAX_PER_IMAGE], y1[0, :MAX_PER_IMAGE],
         x2[0, :MAX_PER_IMAGE], y2[0, :MAX_PER_IMAGE]], axis=-1)
    return out_boxes, out_scores, out_labels


# speedup vs baseline: 28.4710x; 1.0263x over previous
import functools

import jax
import jax.numpy as jnp
from jax.experimental import pallas as pl
from jax.experimental.pallas import tpu as pltpu

IMAGE_SIZE = 512.0
CONFIDENCE_THRESHOLD = 0.01
NMS_THRESHOLD = 0.45
MAX_PER_IMAGE = 100
CENTER_VARIANCE = 0.1
SIZE_VARIANCE = 0.2
NEG = -1e30
CLASS_OFFSET = 4.0 * IMAGE_SIZE

N_PRIORS = 20000
N_CLASSES = 21
C = N_CLASSES - 1
N_BLOCKS = 157
N_PAD = N_BLOCKS * 128
N_KEEP_PAD = 128
BIG_I = 2**30


def _nms_kernel(logits_ref, loc_ref, prior_ref,
                sc_ref, lb_ref, x1_ref, y1_ref, x2_ref, y2_ref,
                cand_ref, colmax_ref, kept_ref):
    logits = logits_ref[...]
    m = jnp.max(logits, axis=0, keepdims=True)
    e = jnp.exp(logits - m)
    probs = e / jnp.sum(e, axis=0, keepdims=True)
    scores = probs[1:, :]

    loc = loc_ref[...]
    pr = prior_ref[...]
    cx = loc[0:1, :] * CENTER_VARIANCE * pr[2:3, :] + pr[0:1, :]
    cy = loc[1:2, :] * CENTER_VARIANCE * pr[3:4, :] + pr[1:2, :]
    w = jnp.exp(loc[2:3, :] * SIZE_VARIANCE) * pr[2:3, :]
    h = jnp.exp(loc[3:4, :] * SIZE_VARIANCE) * pr[3:4, :]
    x1 = (cx - w * 0.5) * IMAGE_SIZE
    y1 = (cy - h * 0.5) * IMAGE_SIZE
    x2 = (cx + w * 0.5) * IMAGE_SIZE
    y2 = (cy + h * 0.5) * IMAGE_SIZE

    S = jnp.where(scores > CONFIDENCE_THRESHOLD, scores, NEG)

    comb = jnp.concatenate(
        [S, x1, y1, x2, y2, jnp.full((8, N_PRIORS), NEG, jnp.float32)], axis=0)
    cand_ref[...] = jnp.transpose(comb)

    colmax_flat = jnp.max(S, axis=0, keepdims=True)
    colmax_flat = jnp.concatenate(
        [colmax_flat, jnp.full((1, N_PAD - N_PRIORS), NEG, jnp.float32)],
        axis=1)
    cm2d = jnp.reshape(colmax_flat, (N_BLOCKS, 128))
    colmax_ref[...] = cm2d

    rowmax = jnp.max(cm2d, axis=1, keepdims=True)
    rowmax_t = jnp.transpose(rowmax)
    rowmax_t = jnp.concatenate(
        [rowmax_t, jnp.full((1, 1024 - N_BLOCKS), NEG, jnp.float32)], axis=1)
    bm0 = jnp.reshape(rowmax_t, (8, 128))

    zf = jnp.zeros((1, N_KEEP_PAD), jnp.float32)
    sc_ref[...] = zf
    lb_ref[...] = jnp.zeros((1, N_KEEP_PAD), jnp.int32)
    x1_ref[...] = zf
    y1_ref[...] = zf
    x2_ref[...] = zf
    y2_ref[...] = zf
    kept_ref[...] = jnp.zeros((8, N_KEEP_PAD), jnp.float32)

    lane32 = jax.lax.broadcasted_iota(jnp.int32, (1, 32), 1)
    lane128 = jax.lax.broadcasted_iota(jnp.int32, (1, N_KEEP_PAD), 1)
    bflat = jax.lax.broadcasted_iota(jnp.int32, (8, 128), 0) * 128 + \
        jax.lax.broadcasted_iota(jnp.int32, (8, 128), 1)

    def cond(carry):
        nkeep, gmax, _ = carry
        return (nkeep < MAX_PER_IMAGE) & (gmax > -1e20)

    def body(carry):
        nkeep, gmax, bm = carry

        b = jnp.min(jnp.where(bm == gmax, bflat, BIG_I))
        crow = colmax_ref[pl.ds(b, 1), :]
        j = jnp.min(jnp.where(crow == gmax, lane128, BIG_I))
        p = b * 128 + j

        row = cand_ref[pl.ds(p, 1), :]
        is_score = lane32 < C
        r_v = jnp.min(jnp.where(is_score & (row == gmax), lane32, BIG_I),
                      axis=1, keepdims=True)

        bx1 = row[:, C:C + 1]
        by1 = row[:, C + 1:C + 2]
        bx2 = row[:, C + 2:C + 3]
        by2 = row[:, C + 3:C + 4]

        off = (r_v + 1).astype(jnp.float32) * CLASS_OFFSET
        bxo1 = bx1 + off
        byo1 = by1 + off
        bxo2 = bx2 + off
        byo2 = by2 + off
        barea = (bxo2 - bxo1) * (byo2 - byo1)

        kox1 = kept_ref[0:1, :]
        koy1 = kept_ref[1:2, :]
        kox2 = kept_ref[2:3, :]
        koy2 = kept_ref[3:4, :]
        karea = kept_ref[4:5, :]
        iw = jnp.maximum(jnp.minimum(kox2, bxo2) - jnp.maximum(kox1, bxo1), 0.0)
        ih = jnp.maximum(jnp.minimum(koy2, byo2) - jnp.maximum(koy1, byo1), 0.0)
        inter = iw * ih
        iou = inter / (karea + barea - inter + 1e-9)
        supp = jnp.any((iou > NMS_THRESHOLD) & (lane128 < nkeep))

        at_k = (lane128 == nkeep) & jnp.logical_not(supp)
        kept_ref[0:1, :] = jnp.where(at_k, bxo1, kox1)
        kept_ref[1:2, :] = jnp.where(at_k, byo1, koy1)
        kept_ref[2:3, :] = jnp.where(at_k, bxo2, kox2)
        kept_ref[3:4, :] = jnp.where(at_k, byo2, koy2)
        kept_ref[4:5, :] = jnp.where(at_k, barea, karea)
        sc_ref[...] = jnp.where(at_k, gmax, sc_ref[...])
        lb_ref[...] = jnp.where(at_k, r_v + 1, lb_ref[...])
        x1_ref[...] = jnp.where(at_k, bx1, x1_ref[...])
        y1_ref[...] = jnp.where(at_k, by1, y1_ref[...])
        x2_ref[...] = jnp.where(at_k, bx2, x2_ref[...])
        y2_ref[...] = jnp.where(at_k, by2, y2_ref[...])
        nkeep = nkeep + jnp.where(supp, 0, 1)

        row_new = jnp.where(lane32 == r_v, NEG, row)
        cand_ref[pl.ds(p, 1), :] = row_new
        head = jnp.max(jnp.where(is_score, row_new, NEG), axis=1,
                       keepdims=True)
        crow_new = jnp.where(lane128 == j, head, crow)
        colmax_ref[pl.ds(b, 1), :] = crow_new
        bm = jnp.where(bflat == b, jnp.max(crow_new), bm)
        gmax = jnp.max(bm)

        return (nkeep, gmax, bm)

    carry = (jnp.int32(0), jnp.max(bm0), bm0)
    jax.lax.while_loop(cond, body, carry)


@jax.jit
def kernel(cls_logits, bbox_pred, priors):
    logits_t = jnp.transpose(cls_logits[0])
    loc_t = jnp.transpose(bbox_pred[0])
    prior_t = jnp.transpose(priors)

    out_shapes = (
        jax.ShapeDtypeStruct((1, N_KEEP_PAD), jnp.float32),
        jax.ShapeDtypeStruct((1, N_KEEP_PAD), jnp.int32),
        jax.ShapeDtypeStruct((1, N_KEEP_PAD), jnp.float32),
        jax.ShapeDtypeStruct((1, N_KEEP_PAD), jnp.float32),
        jax.ShapeDtypeStruct((1, N_KEEP_PAD), jnp.float32),
        jax.ShapeDtypeStruct((1, N_KEEP_PAD), jnp.float32),
    )
    sc, lb, x1, y1, x2, y2 = pl.pallas_call(
        _nms_kernel,
        out_shape=out_shapes,
        scratch_shapes=[
            pltpu.VMEM((N_PRIORS, 32), jnp.float32),
            pltpu.VMEM((N_BLOCKS, 128), jnp.float32),
            pltpu.VMEM((8, N_KEEP_PAD), jnp.float32),
        ],
    )(logits_t, loc_t, prior_t)

    out_scores = sc[0, :MAX_PER_IMAGE]
    out_labels = lb[0, :MAX_PER_IMAGE]
    out_boxes = jnp.stack(
        [x1[0, :MAX_PER_IMAGE], y1[0, :MAX_PER_IMAGE],
         x2[0, :MAX_PER_IMAGE], y2[0, :MAX_PER_IMAGE]], axis=-1)
    return out_boxes, out_scores, out_labels


# speedup vs baseline: 38.8165x; 1.3634x over previous
import jax
import jax.numpy as jnp
from jax.experimental import pallas as pl
from jax.experimental.pallas import tpu as pltpu

IMAGE_SIZE = 512.0
CONFIDENCE_THRESHOLD = 0.01
NMS_THRESHOLD = 0.45
MAX_PER_IMAGE = 100
CENTER_VARIANCE = 0.1
SIZE_VARIANCE = 0.2
NEG = -1e30
CLASS_OFFSET = 4.0 * IMAGE_SIZE

N_PRIORS = 20000
N_CLASSES = 21
C = N_CLASSES - 1
N_BLOCKS = 157
N_PAD = N_BLOCKS * 128
N_KEEP_PAD = 128
BIG_I = 2**30


def _nms_kernel(logits_ref, loc_ref, prior_ref, out_ref, cand_ref):
    logits = logits_ref[...]
    m = jnp.max(logits, axis=0, keepdims=True)
    e = jnp.exp(logits - m)
    probs = e / jnp.sum(e, axis=0, keepdims=True)
    scores = probs[1:, :]

    loc = loc_ref[...]
    pr = prior_ref[...]
    cx = loc[0:1, :] * CENTER_VARIANCE * pr[2:3, :] + pr[0:1, :]
    cy = loc[1:2, :] * CENTER_VARIANCE * pr[3:4, :] + pr[1:2, :]
    w = jnp.exp(loc[2:3, :] * SIZE_VARIANCE) * pr[2:3, :]
    h = jnp.exp(loc[3:4, :] * SIZE_VARIANCE) * pr[3:4, :]
    x1 = (cx - w * 0.5) * IMAGE_SIZE
    y1 = (cy - h * 0.5) * IMAGE_SIZE
    x2 = (cx + w * 0.5) * IMAGE_SIZE
    y2 = (cy + h * 0.5) * IMAGE_SIZE

    S = jnp.where(scores > CONFIDENCE_THRESHOLD, scores, NEG)

    comb = jnp.concatenate(
        [S, x1, y1, x2, y2, jnp.full((8, N_PRIORS), NEG, jnp.float32)], axis=0)
    cand_ref[...] = jnp.transpose(comb)

    colmax_flat = jnp.max(S, axis=0, keepdims=True)
    colmax_flat = jnp.concatenate(
        [colmax_flat, jnp.full((1, N_PAD - N_PRIORS), NEG, jnp.float32)],
        axis=1)
    cm0 = jnp.reshape(colmax_flat, (N_BLOCKS, 128))

    lane32 = jax.lax.broadcasted_iota(
        jnp.int32, (1, 32), 1).astype(jnp.float32)
    lane128 = jax.lax.broadcasted_iota(
        jnp.int32, (1, N_KEEP_PAD), 1).astype(jnp.float32)
    sub8 = jax.lax.broadcasted_iota(jnp.int32, (8, N_KEEP_PAD), 0)
    fidx = (jax.lax.broadcasted_iota(jnp.int32, (N_BLOCKS, 128), 0) * 128 +
            jax.lax.broadcasted_iota(jnp.int32, (N_BLOCKS, 128), 1)
            ).astype(jnp.float32)
    is_score = lane32 < float(C)

    zero8 = jnp.zeros((8, N_KEEP_PAD), jnp.float32)

    def _nkeep(kept):
        return jnp.sum(jnp.where(kept[4:5, :] > 0.0, 1.0, 0.0),
                       axis=1, keepdims=True)

    def _gmax(cm):
        return jnp.max(jnp.max(cm, axis=0, keepdims=True),
                       axis=1, keepdims=True)

    def cond(carry):
        cm, kept, _ = carry
        nk = _nkeep(kept)
        gm = _gmax(cm)
        return (nk[0, 0] < float(MAX_PER_IMAGE)) & (gm[0, 0] > -1e20)

    def body(carry):
        cm, kept, outv = carry
        nkeep = _nkeep(kept)
        gmax = _gmax(cm)

        t = jnp.min(jnp.where(cm == gmax, fidx, 1e9),
                    axis=0, keepdims=True)
        p_v = jnp.min(t, axis=1, keepdims=True)
        p = p_v[0, 0].astype(jnp.int32)

        row = cand_ref[pl.ds(p, 1), :]
        r_v = jnp.min(jnp.where(is_score & (row == gmax), lane32, 1e9),
                      axis=1, keepdims=True)

        bx1 = row[:, C:C + 1]
        by1 = row[:, C + 1:C + 2]
        bx2 = row[:, C + 2:C + 3]
        by2 = row[:, C + 3:C + 4]

        off = (r_v + 1.0) * CLASS_OFFSET
        bxo1 = bx1 + off
        byo1 = by1 + off
        bxo2 = bx2 + off
        byo2 = by2 + off
        barea = (bxo2 - bxo1) * (byo2 - byo1)

        kox1 = kept[0:1, :]
        koy1 = kept[1:2, :]
        kox2 = kept[2:3, :]
        koy2 = kept[3:4, :]
        karea = kept[4:5, :]
        iw = jnp.maximum(jnp.minimum(kox2, bxo2) - jnp.maximum(kox1, bxo1),
                         0.0)
        ih = jnp.maximum(jnp.minimum(koy2, byo2) - jnp.maximum(koy1, byo1),
                         0.0)
        inter = iw * ih
        iou = inter / (karea + barea - inter + 1e-9)
        hit = (iou > NMS_THRESHOLD) & (lane128 < nkeep)
        suppf = jnp.max(jnp.where(hit, 1.0, 0.0),
                        axis=1, keepdims=True)

        at_lane = jnp.where(lane128 == nkeep, 1.0, 0.0) * (1.0 - suppf)
        at_mask = at_lane > 0.5
        kval = jnp.where(sub8 == 0, bxo1,
                         jnp.where(sub8 == 1, byo1,
                                   jnp.where(sub8 == 2, bxo2,
                                             jnp.where(sub8 == 3, byo2,
                                                       barea))))
        kept_new = jnp.where(at_mask & (sub8 < 5), kval, kept)

        oval = jnp.where(sub8 == 0, gmax,
                         jnp.where(sub8 == 1, r_v + 1.0,
                                   jnp.where(sub8 == 2, bx1,
                                             jnp.where(sub8 == 3, by1,
                                                       jnp.where(sub8 == 4,
                                                                 bx2, by2)))))
        outv_new = jnp.where(at_mask & (sub8 < 6), oval, outv)

        row_new = jnp.where(lane32 == r_v, NEG, row)
        cand_ref[pl.ds(p, 1), :] = row_new
        head = jnp.max(jnp.where(is_score, row_new, NEG),
                       axis=1, keepdims=True)
        cm_new = jnp.where(fidx == p_v, head, cm)

        return (cm_new, kept_new, outv_new)

    final = jax.lax.while_loop(cond, body, (cm0, zero8, zero8))
    out_ref[...] = final[2]


@jax.jit
def kernel(cls_logits, bbox_pred, priors):
    logits_t = jnp.transpose(cls_logits[0])
    loc_t = jnp.transpose(bbox_pred[0])
    prior_t = jnp.transpose(priors)

    out = pl.pallas_call(
        _nms_kernel,
        out_shape=jax.ShapeDtypeStruct((8, N_KEEP_PAD), jnp.float32),
        scratch_shapes=[pltpu.VMEM((N_PRIORS, 32), jnp.float32)],
    )(logits_t, loc_t, prior_t)

    out_scores = out[0, :MAX_PER_IMAGE]
    out_labels = out[1, :MAX_PER_IMAGE].astype(jnp.int32)
    out_boxes = jnp.stack(
        [out[2, :MAX_PER_IMAGE], out[3, :MAX_PER_IMAGE],
         out[4, :MAX_PER_IMAGE], out[5, :MAX_PER_IMAGE]], axis=-1)
    return out_boxes, out_scores, out_labels


# speedup vs baseline: 44.6198x; 1.1495x over previous
import jax
import jax.numpy as jnp
from jax.experimental import pallas as pl
from jax.experimental.pallas import tpu as pltpu

IMAGE_SIZE = 512.0
CONFIDENCE_THRESHOLD = 0.01
NMS_THRESHOLD = 0.45
MAX_PER_IMAGE = 100
CENTER_VARIANCE = 0.1
SIZE_VARIANCE = 0.2
NEG = -1e30
CLASS_OFFSET = 4.0 * IMAGE_SIZE

N_PRIORS = 20000
N_CLASSES = 21
C = N_CLASSES - 1
N_BLOCKS = 157
N_PAD = N_BLOCKS * 128
N_KEEP_PAD = 128
BIG_I = 2**30


def _nms_kernel(logits_ref, loc_ref, prior_ref, out_ref, cand_ref):
    logits = logits_ref[...]
    m = jnp.max(logits, axis=0, keepdims=True)
    e = jnp.exp(logits - m)
    probs = e / jnp.sum(e, axis=0, keepdims=True)
    scores = probs[1:, :]

    loc = loc_ref[...]
    pr = prior_ref[...]
    cx = loc[0:1, :] * CENTER_VARIANCE * pr[2:3, :] + pr[0:1, :]
    cy = loc[1:2, :] * CENTER_VARIANCE * pr[3:4, :] + pr[1:2, :]
    w = jnp.exp(loc[2:3, :] * SIZE_VARIANCE) * pr[2:3, :]
    h = jnp.exp(loc[3:4, :] * SIZE_VARIANCE) * pr[3:4, :]
    x1 = (cx - w * 0.5) * IMAGE_SIZE
    y1 = (cy - h * 0.5) * IMAGE_SIZE
    x2 = (cx + w * 0.5) * IMAGE_SIZE
    y2 = (cy + h * 0.5) * IMAGE_SIZE

    S = jnp.where(scores > CONFIDENCE_THRESHOLD, scores, NEG)

    comb = jnp.concatenate(
        [S, x1, y1, x2, y2, jnp.full((8, N_PRIORS), NEG, jnp.float32)], axis=0)
    cand_ref[...] = jnp.transpose(comb)

    colmax_flat = jnp.max(S, axis=0, keepdims=True)
    colmax_flat = jnp.concatenate(
        [colmax_flat, jnp.full((1, N_PAD - N_PRIORS), NEG, jnp.float32)],
        axis=1)
    cm0 = jnp.reshape(colmax_flat, (N_BLOCKS, 128))

    lane32 = jax.lax.broadcasted_iota(
        jnp.int32, (1, 32), 1).astype(jnp.float32)
    lane128 = jax.lax.broadcasted_iota(
        jnp.int32, (1, N_KEEP_PAD), 1).astype(jnp.float32)
    sub8 = jax.lax.broadcasted_iota(jnp.int32, (8, N_KEEP_PAD), 0)
    fidx = (jax.lax.broadcasted_iota(jnp.int32, (N_BLOCKS, 128), 0) * 128 +
            jax.lax.broadcasted_iota(jnp.int32, (N_BLOCKS, 128), 1)
            ).astype(jnp.float32)
    is_score = lane32 < float(C)

    zero8 = jnp.zeros((8, N_KEEP_PAD), jnp.float32)
    zerolane = jnp.zeros((1, N_KEEP_PAD), jnp.float32)
    nextoh0 = jnp.where(lane128 == 0.0, 1.0, 0.0)
    oh_last = jnp.where(lane128 == float(MAX_PER_IMAGE - 1), 1.0, 0.0)

    def _gmax(cm):
        return jnp.max(jnp.max(cm, axis=0, keepdims=True),
                       axis=1, keepdims=True)

    def cond(carry):
        return carry[0] > -1e20

    def body(carry):
        gmax, cm, kept, valid, nextoh, outv = carry

        t = jnp.min(jnp.where(cm == gmax, fidx, 1e9),
                    axis=0, keepdims=True)
        p_v = jnp.min(t, axis=1, keepdims=True)
        p = p_v[0, 0].astype(jnp.int32)

        row = cand_ref[pl.ds(p, 1), :]
        r_v = jnp.min(jnp.where(is_score & (row == gmax), lane32, 1e9),
                      axis=1, keepdims=True)

        bx1 = row[:, C:C + 1]
        by1 = row[:, C + 1:C + 2]
        bx2 = row[:, C + 2:C + 3]
        by2 = row[:, C + 3:C + 4]

        off = (r_v + 1.0) * CLASS_OFFSET
        bxo1 = bx1 + off
        byo1 = by1 + off
        bxo2 = bx2 + off
        byo2 = by2 + off
        barea = (bxo2 - bxo1) * (byo2 - byo1)

        kox1 = kept[0:1, :]
        koy1 = kept[1:2, :]
        kox2 = kept[2:3, :]
        koy2 = kept[3:4, :]
        karea = kept[4:5, :]
        iw = jnp.maximum(jnp.minimum(kox2, bxo2) - jnp.maximum(kox1, bxo1),
                         0.0)
        ih = jnp.maximum(jnp.minimum(koy2, byo2) - jnp.maximum(koy1, byo1),
                         0.0)
        inter = iw * ih
        iou = inter / (karea + barea - inter + 1e-9)
        hit = (iou > NMS_THRESHOLD) & (valid > 0.5)
        suppf = jnp.max(jnp.where(hit, 1.0, 0.0),
                        axis=1, keepdims=True)

        at_lane = nextoh * (1.0 - suppf)
        at_mask = at_lane > 0.5
        kval = jnp.where(sub8 == 0, bxo1,
                         jnp.where(sub8 == 1, byo1,
                                   jnp.where(sub8 == 2, bxo2,
                                             jnp.where(sub8 == 3, byo2,
                                                       barea))))
        kept_new = jnp.where(at_mask & (sub8 < 5), kval, kept)

        oval = jnp.where(sub8 == 0, gmax,
                         jnp.where(sub8 == 1, r_v + 1.0,
                                   jnp.where(sub8 == 2, bx1,
                                             jnp.where(sub8 == 3, by1,
                                                       jnp.where(sub8 == 4,
                                                                 bx2, by2)))))
        outv_new = jnp.where(at_mask & (sub8 < 6), oval, outv)

        valid_new = valid + at_lane
        shifted = jnp.concatenate([jnp.zeros((1, 1), jnp.float32),
                                   nextoh[:, :N_KEEP_PAD - 1]], axis=1)
        nextoh_new = shifted * (1.0 - suppf) + nextoh * suppf

        row_new = jnp.where(lane32 == r_v, NEG, row)
        cand_ref[pl.ds(p, 1), :] = row_new
        head = jnp.max(jnp.where(is_score, row_new, NEG),
                       axis=1, keepdims=True)
        cm_new = jnp.where(fidx == p_v, head, cm)

        gmax_new = _gmax(cm_new)
        full = jnp.max(valid_new * oh_last, axis=1, keepdims=True)
        gm_next = jnp.where(full > 0.5, -1e30, gmax_new)[0, 0]

        return (gm_next, cm_new, kept_new, valid_new, nextoh_new, outv_new)

    gm0 = _gmax(cm0)[0, 0]
    final = jax.lax.while_loop(
        cond, body, (gm0, cm0, zero8, zerolane, nextoh0, zero8))
    out_ref[...] = final[5]


@jax.jit
def kernel(cls_logits, bbox_pred, priors):
    logits_t = jnp.transpose(cls_logits[0])
    loc_t = jnp.transpose(bbox_pred[0])
    prior_t = jnp.transpose(priors)

    out = pl.pallas_call(
        _nms_kernel,
        out_shape=jax.ShapeDtypeStruct((8, N_KEEP_PAD), jnp.float32),
        scratch_shapes=[pltpu.VMEM((N_PRIORS, 32), jnp.float32)],
    )(logits_t, loc_t, prior_t)

    out_scores = out[0, :MAX_PER_IMAGE]
    out_labels = out[1, :MAX_PER_IMAGE].astype(jnp.int32)
    out_boxes = jnp.stack(
        [out[2, :MAX_PER_IMAGE], out[3, :MAX_PER_IMAGE],
         out[4, :MAX_PER_IMAGE], out[5, :MAX_PER_IMAGE]], axis=-1)
    return out_boxes, out_scores, out_labels


# speedup vs baseline: 47.5655x; 1.0660x over previous
import jax
import jax.numpy as jnp
from jax.experimental import pallas as pl
from jax.experimental.pallas import tpu as pltpu

IMAGE_SIZE = 512.0
CONFIDENCE_THRESHOLD = 0.01
NMS_THRESHOLD = 0.45
MAX_PER_IMAGE = 100
CENTER_VARIANCE = 0.1
SIZE_VARIANCE = 0.2
NEG = -1e30
CLASS_OFFSET = 4.0 * IMAGE_SIZE

N_PRIORS = 20000
N_CLASSES = 21
C = N_CLASSES - 1
N_BLOCKS = 157
N_PAD = N_BLOCKS * 128
N_KEEP_PAD = 128
BIG_I = 2**30


def _nms_kernel(logits_ref, loc_ref, prior_ref, out_ref, cand_ref):
    logits = logits_ref[...]
    m = jnp.max(logits, axis=0, keepdims=True)
    e = jnp.exp(logits - m)
    probs = e / jnp.sum(e, axis=0, keepdims=True)
    scores = probs[1:, :]

    loc = loc_ref[...]
    pr = prior_ref[...]
    cx = loc[0:1, :] * CENTER_VARIANCE * pr[2:3, :] + pr[0:1, :]
    cy = loc[1:2, :] * CENTER_VARIANCE * pr[3:4, :] + pr[1:2, :]
    w = jnp.exp(loc[2:3, :] * SIZE_VARIANCE) * pr[2:3, :]
    h = jnp.exp(loc[3:4, :] * SIZE_VARIANCE) * pr[3:4, :]
    x1 = (cx - w * 0.5) * IMAGE_SIZE
    y1 = (cy - h * 0.5) * IMAGE_SIZE
    x2 = (cx + w * 0.5) * IMAGE_SIZE
    y2 = (cy + h * 0.5) * IMAGE_SIZE

    S = jnp.where(scores > CONFIDENCE_THRESHOLD, scores, NEG)

    comb = jnp.concatenate(
        [S, x1, y1, x2, y2, jnp.full((8, N_PRIORS), NEG, jnp.float32)], axis=0)
    cand_ref[...] = jnp.transpose(comb)

    colmax_flat = jnp.max(S, axis=0, keepdims=True)
    colmax_flat = jnp.concatenate(
        [colmax_flat, jnp.full((1, N_PAD - N_PRIORS), NEG, jnp.float32)],
        axis=1)
    cm0 = jnp.reshape(colmax_flat, (N_BLOCKS, 128))

    lane32 = jax.lax.broadcasted_iota(
        jnp.int32, (1, 32), 1).astype(jnp.float32)
    lane128 = jax.lax.broadcasted_iota(
        jnp.int32, (1, N_KEEP_PAD), 1).astype(jnp.float32)
    sub8 = jax.lax.broadcasted_iota(jnp.int32, (8, N_KEEP_PAD), 0)
    fidx = (jax.lax.broadcasted_iota(jnp.int32, (N_BLOCKS, 128), 0) * 128 +
            jax.lax.broadcasted_iota(jnp.int32, (N_BLOCKS, 128), 1)
            ).astype(jnp.float32)
    is_score = lane32 < float(C)

    zero8 = jnp.zeros((8, N_KEEP_PAD), jnp.float32)
    zerolane = jnp.zeros((1, N_KEEP_PAD), jnp.float32)
    nextoh0 = jnp.where(lane128 == 0.0, 1.0, 0.0)
    oh_last = jnp.where(lane128 == float(MAX_PER_IMAGE - 1), 1.0, 0.0)

    def _gmax(cm):
        return jnp.max(jnp.max(cm, axis=0, keepdims=True),
                       axis=1, keepdims=True)

    def _pfind(cm, gm):
        t = jnp.min(jnp.where(cm == gm, fidx, 1e9),
                    axis=0, keepdims=True)
        return jnp.min(t, axis=1, keepdims=True)

    def cond(carry):
        return carry[0] > -0.5

    def body(carry):
        p_code, cm, kept, valid, nextoh, outv = carry
        p = p_code.astype(jnp.int32)

        row = cand_ref[pl.ds(p, 1), :]
        rs = jnp.where(is_score, row, NEG)
        gmax = jnp.max(rs, axis=1, keepdims=True)
        r_v = jnp.min(jnp.where(rs == gmax, lane32, 1e9),
                      axis=1, keepdims=True)

        bx1 = row[:, C:C + 1]
        by1 = row[:, C + 1:C + 2]
        bx2 = row[:, C + 2:C + 3]
        by2 = row[:, C + 3:C + 4]

        off = (r_v + 1.0) * CLASS_OFFSET
        bxo1 = bx1 + off
        byo1 = by1 + off
        bxo2 = bx2 + off
        byo2 = by2 + off
        barea = (bxo2 - bxo1) * (byo2 - byo1)

        kox1 = kept[0:1, :]
        koy1 = kept[1:2, :]
        kox2 = kept[2:3, :]
        koy2 = kept[3:4, :]
        karea = kept[4:5, :]
        iw = jnp.maximum(jnp.minimum(kox2, bxo2) - jnp.maximum(kox1, bxo1),
                         0.0)
        ih = jnp.maximum(jnp.minimum(koy2, byo2) - jnp.maximum(koy1, byo1),
                         0.0)
        inter = iw * ih
        iou = inter / (karea + barea - inter + 1e-9)
        hit = (iou > NMS_THRESHOLD) & (valid > 0.5)
        suppf = jnp.max(jnp.where(hit, 1.0, 0.0),
                        axis=1, keepdims=True)

        at_lane = nextoh * (1.0 - suppf)
        at_mask = at_lane > 0.5
        kval = jnp.where(sub8 == 0, bxo1,
                         jnp.where(sub8 == 1, byo1,
                                   jnp.where(sub8 == 2, bxo2,
                                             jnp.where(sub8 == 3, byo2,
                                                       barea))))
        kept_new = jnp.where(at_mask & (sub8 < 5), kval, kept)

        oval = jnp.where(sub8 == 0, gmax,
                         jnp.where(sub8 == 1, r_v + 1.0,
                                   jnp.where(sub8 == 2, bx1,
                                             jnp.where(sub8 == 3, by1,
                                                       jnp.where(sub8 == 4,
                                                                 bx2, by2)))))
        outv_new = jnp.where(at_mask & (sub8 < 6), oval, outv)

        valid_new = valid + at_lane
        shifted = jnp.concatenate([jnp.zeros((1, 1), jnp.float32),
                                   nextoh[:, :N_KEEP_PAD - 1]], axis=1)
        nextoh_new = shifted * (1.0 - suppf) + nextoh * suppf

        row_new = jnp.where(lane32 == r_v, NEG, row)
        cand_ref[pl.ds(p, 1), :] = row_new
        head = jnp.max(jnp.where(is_score, row_new, NEG),
                       axis=1, keepdims=True)
        cm_new = jnp.where(fidx == p_code, head, cm)

        gmax_new = _gmax(cm_new)
        p_next = _pfind(cm_new, gmax_new)
        full = jnp.max(valid_new * oh_last, axis=1, keepdims=True)
        stop = (full > 0.5) | (gmax_new < -1e20)
        p_code_next = jnp.where(stop, -1.0, p_next)[0, 0]

        return (p_code_next, cm_new, kept_new, valid_new, nextoh_new,
                outv_new)

    gm0 = _gmax(cm0)
    p_code0 = jnp.where(gm0 < -1e20, -1.0, _pfind(cm0, gm0))[0, 0]
    final = jax.lax.while_loop(
        cond, body, (p_code0, cm0, zero8, zerolane, nextoh0, zero8))
    out_ref[...] = final[5]


@jax.jit
def kernel(cls_logits, bbox_pred, priors):
    logits_t = jnp.transpose(cls_logits[0])
    loc_t = jnp.transpose(bbox_pred[0])
    prior_t = jnp.transpose(priors)

    out = pl.pallas_call(
        _nms_kernel,
        out_shape=jax.ShapeDtypeStruct((8, N_KEEP_PAD), jnp.float32),
        scratch_shapes=[pltpu.VMEM((N_PRIORS, 32), jnp.float32)],
    )(logits_t, loc_t, prior_t)

    out_scores = out[0, :MAX_PER_IMAGE]
    out_labels = out[1, :MAX_PER_IMAGE].astype(jnp.int32)
    out_boxes = jnp.stack(
        [out[2, :MAX_PER_IMAGE], out[3, :MAX_PER_IMAGE],
         out[4, :MAX_PER_IMAGE], out[5, :MAX_PER_IMAGE]], axis=-1)
    return out_boxes, out_scores, out_labels
